# baseline (speedup 1.0000x reference)
import math

import numpy as np

import jax
import jax.numpy as jnp
from jax.experimental import pallas as pl

K = 1000
S = 8
D = 1024
B = 4096
BETA = 5.0
ALPHA = 2.0

_XSCALE = BETA * math.log2(math.e)
_OSCALE = ALPHA * math.exp(-BETA)

_BB = 2048
_BK = 256
_BKS = _BK * S
_NK = -(-K // _BK)
_KLAST = K - (_NK - 1) * _BK
_SUB = 2
_RS = _BKS // _SUB

_G = np.kron(np.eye(_BK, dtype=np.float32), np.ones((S, 1), np.float32))


def _tda_kernel(x_ref, m_ref, g_ref, o_ref):
    j = pl.program_id(1)
    xb = (x_ref[...] * _XSCALE).astype(jnp.bfloat16)
    mflat = m_ref[...].reshape(_BKS, D)
    limit = (K - j * _BK) * S
    row = jax.lax.broadcasted_iota(jnp.int32, (_RS, D), 0)
    o = None
    for c in range(_SUB):
        r0 = c * _RS
        mb = jnp.where(row + r0 < limit, mflat[r0:r0 + _RS], 0.0)
        mb = mb.astype(jnp.bfloat16)
        a = jax.lax.dot_general(
            xb, mb,
            dimension_numbers=(((1,), (1,)), ((), ())),
            preferred_element_type=jnp.float32,
        )
        e = jnp.exp2(a).astype(jnp.bfloat16)
        oc = jax.lax.dot_general(
            e, g_ref[r0:r0 + _RS, :],
            dimension_numbers=(((1,), (0,)), ((), ())),
            preferred_element_type=jnp.float32,
        )
        o = oc if o is None else o + oc
    o_ref[...] = o * _OSCALE


def kernel(x, memory):
    grid = (B // _BB, _NK)
    return pl.pallas_call(
        _tda_kernel,
        grid=grid,
        in_specs=[
            pl.BlockSpec((_BB, D), lambda i, j: (i, 0)),
            pl.BlockSpec((_BK, S, D), lambda i, j: (j, 0, 0)),
            pl.BlockSpec((_BKS, _BK), lambda i, j: (0, 0)),
        ],
        out_specs=pl.BlockSpec((_BB, _BK), lambda i, j: (i, j)),
        out_shape=jax.ShapeDtypeStruct((B, K), jnp.float32),
    )(x, memory, jnp.asarray(_G, dtype=jnp.bfloat16))


# speedup vs baseline: 1.1468x; 1.1468x over previous
import math

import numpy as np

import jax
import jax.numpy as jnp
from jax.experimental import pallas as pl

K = 1000
S = 8
D = 1024
B = 4096
BETA = 5.0
ALPHA = 2.0

_XSCALE = BETA * math.log2(math.e)
_OSCALE = ALPHA * math.exp(-BETA)

_BB = 2048
_BK = 256
_BKS = _BK * S
_NK = -(-K // _BK)
_KLAST = K - (_NK - 1) * _BK
_SUB = 4
_RS = _BKS // _SUB

_G = np.kron(np.eye(_BK, dtype=np.float32), np.ones((S, 1), np.float32))


def _tda_kernel(x_ref, m_ref, g_ref, o_ref):
    j = pl.program_id(1)
    xb = (x_ref[...] * _XSCALE).astype(jnp.bfloat16)
    mflat = m_ref[...].reshape(_BKS, D)
    limit = (K - j * _BK) * S
    row = jax.lax.broadcasted_iota(jnp.int32, (_RS, D), 0)
    o = None
    for c in range(_SUB):
        r0 = c * _RS
        mb = jnp.where(row + r0 < limit, mflat[r0:r0 + _RS], 0.0)
        mb = mb.astype(jnp.bfloat16)
        a = jax.lax.dot_general(
            xb, mb,
            dimension_numbers=(((1,), (1,)), ((), ())),
            preferred_element_type=jnp.float32,
        )
        e = jnp.exp2(a).astype(jnp.bfloat16)
        oc = jax.lax.dot_general(
            e, g_ref[r0:r0 + _RS, :],
            dimension_numbers=(((1,), (0,)), ((), ())),
            preferred_element_type=jnp.float32,
        )
        o = oc if o is None else o + oc
    o_ref[...] = o * _OSCALE


def kernel(x, memory):
    grid = (B // _BB, _NK)
    return pl.pallas_call(
        _tda_kernel,
        grid=grid,
        in_specs=[
            pl.BlockSpec((_BB, D), lambda i, j: (i, 0)),
            pl.BlockSpec((_BK, S, D), lambda i, j: (j, 0, 0)),
            pl.BlockSpec((_BKS, _BK), lambda i, j: (0, 0)),
        ],
        out_specs=pl.BlockSpec((_BB, _BK), lambda i, j: (i, j)),
        out_shape=jax.ShapeDtypeStruct((B, K), jnp.float32),
    )(x, memory, jnp.asarray(_G, dtype=jnp.bfloat16))


# speedup vs baseline: 1.1473x; 1.0005x over previous
import math

import numpy as np

import jax
import jax.numpy as jnp
from jax.experimental import pallas as pl

K = 1000
S = 8
D = 1024
B = 4096
BETA = 5.0
ALPHA = 2.0

_XSCALE = BETA * math.log2(math.e)
_OSCALE = ALPHA * math.exp(-BETA)

_BB = 2048
_BK = 256
_BKS = _BK * S
_NK = -(-K // _BK)
_SUB = 4
_RS = _BKS // _SUB

_G = np.kron(np.eye(_BK, dtype=np.float32), np.ones((S, 1), np.float32))


def _tda_kernel(x_ref, m_ref, g_ref, o_ref):
    j = pl.program_id(1)
    xb = (x_ref[...] * _XSCALE).astype(jnp.bfloat16)
    mflat = m_ref[...].reshape(_BKS, D)
    limit = (K - j * _BK) * S
    row = jax.lax.broadcasted_iota(jnp.int32, (_RS, D), 0)
    o = None
    for c in range(_SUB):
        r0 = c * _RS
        mb = jnp.where(row + r0 < limit, mflat[r0:r0 + _RS], 0.0)
        mb = mb.astype(jnp.bfloat16)
        a = jax.lax.dot_general(
            xb, mb,
            dimension_numbers=(((1,), (1,)), ((), ())),
            preferred_element_type=jnp.float32,
        )
        e = jnp.exp2(a).astype(jnp.bfloat16)
        oc = jax.lax.dot_general(
            e, g_ref[r0:r0 + _RS, :],
            dimension_numbers=(((1,), (0,)), ((), ())),
            preferred_element_type=jnp.float32,
        )
        o = oc if o is None else o + oc
    o_ref[...] = o * _OSCALE


def kernel(x, memory):
    grid = (B // _BB, _NK)
    return pl.pallas_call(
        _tda_kernel,
        grid=grid,
        in_specs=[
            pl.BlockSpec((_BB, D), lambda i, j: (i, 0)),
            pl.BlockSpec((_BK, S, D), lambda i, j: (j, 0, 0)),
            pl.BlockSpec((_BKS, _BK), lambda i, j: (0, 0)),
        ],
        out_specs=pl.BlockSpec((_BB, _BK), lambda i, j: (i, j)),
        out_shape=jax.ShapeDtypeStruct((B, K), jnp.float32),
    )(x, memory, jnp.asarray(_G, dtype=jnp.bfloat16))
